# Initial kernel scaffold; baseline (speedup 1.0000x reference)
import jax
import jax.numpy as jnp
from jax.experimental import pallas as pl


def kernel(x, edge_index, W, b, ln_gamma, ln_beta):
    raise NotImplementedError("write your pallas kernel here")


# speedup vs baseline: 8.9868x; 8.9868x over previous
import jax
import jax.numpy as jnp
from jax import lax
from jax.experimental import pallas as pl
from jax.experimental.pallas import tpu as pltpu
from jax.experimental.pallas import tpu_sc as plsc

N = 10000
E = 160000
D = 256
DH = D // 2
EPS = 1e-5

NS = 16
L = 16
EPT = E // NS
K = 80
ECH = EPT // K
NP = 10240
RPT = NP // NS
KR = 128
RCH = RPT // KR
NA = N + L

_mesh = plsc.VectorSubcoreMesh(core_axis_name="c", subcore_axis_name="s")


def _deg_body(dst_hbm, rid_hbm, ones_hbm, zer_hbm, out_hbm, idx_v, ridx_v,
              ones_v, zbuf_v, gbuf_v, hist_sh):
    cid = lax.axis_index("c")
    sid = lax.axis_index("s")

    @pl.when(cid == 0)
    def _():
        pltpu.sync_copy(ones_hbm, ones_v)
        pltpu.sync_copy(zer_hbm, zbuf_v)

        @pl.loop(0, RCH)
        def _(j):
            off = pl.multiple_of(sid * RPT + j * KR, 8)
            pltpu.sync_copy(rid_hbm.at[pl.ds(off, KR)], ridx_v)
            pltpu.sync_copy(zbuf_v, hist_sh.at[ridx_v])

    plsc.subcore_barrier()

    @pl.when(cid == 0)
    def _():
        @pl.loop(0, ECH)
        def _(i):
            pltpu.sync_copy(dst_hbm.at[pl.ds(sid * EPT + i * K, K)], idx_v)
            pltpu.sync_copy(ones_v, hist_sh.at[idx_v], add=True)

    plsc.subcore_barrier()

    @pl.when(cid == 0)
    def _():
        @pl.loop(0, RCH)
        def _(j):
            off = pl.multiple_of(sid * RPT + j * KR, 8)
            pltpu.sync_copy(rid_hbm.at[pl.ds(off, KR)], ridx_v)
            pltpu.sync_copy(hist_sh.at[ridx_v], gbuf_v)
            pltpu.sync_copy(gbuf_v, out_hbm.at[pl.ds(off, KR)])


_deg_call = pl.kernel(
    _deg_body,
    out_type=jax.ShapeDtypeStruct((NP, L), jnp.float32),
    mesh=_mesh,
    scratch_types=[
        pltpu.VMEM((K,), jnp.int32),
        pltpu.VMEM((KR,), jnp.int32),
        pltpu.VMEM((K, L), jnp.float32),
        pltpu.VMEM((KR, L), jnp.float32),
        pltpu.VMEM((KR, L), jnp.float32),
        pltpu.VMEM_SHARED((NA, L), jnp.float32),
    ],
)


def _seg_body(gl_hbm, gr_hbm, src_hbm, dst_hbm, rid_hbm, zer_hbm,
              outl_hbm, outr_hbm,
              src_v, dst_v, ridx_v, rows_v, zbuf_v, gbuf_v, acc_sh, sem):
    cid = lax.axis_index("c")
    sid = lax.axis_index("s")

    def run(g_hbm, out_hbm):
        pltpu.sync_copy(zer_hbm, zbuf_v)

        @pl.loop(0, RCH)
        def _(j):
            off = pl.multiple_of(sid * RPT + j * KR, 8)
            pltpu.sync_copy(rid_hbm.at[pl.ds(off, KR)], ridx_v)
            pltpu.sync_copy(zbuf_v, acc_sh.at[ridx_v])

        plsc.subcore_barrier()

        @pl.loop(0, ECH)
        def _(i):
            eb = sid * EPT + i * K
            pltpu.sync_copy(src_hbm.at[pl.ds(eb, K)], src_v)
            pltpu.sync_copy(dst_hbm.at[pl.ds(eb, K)], dst_v)
            pltpu.async_copy(g_hbm.at[src_v], rows_v, sem).wait()
            pltpu.sync_copy(rows_v, acc_sh.at[dst_v], add=True)

        plsc.subcore_barrier()

        @pl.loop(0, RCH)
        def _(j):
            off = pl.multiple_of(sid * RPT + j * KR, 8)
            pltpu.sync_copy(rid_hbm.at[pl.ds(off, KR)], ridx_v)
            pltpu.sync_copy(acc_sh.at[ridx_v], gbuf_v)
            pltpu.sync_copy(gbuf_v, out_hbm.at[pl.ds(off, KR)])

    @pl.when(cid == 0)
    def _():
        run(gl_hbm, outl_hbm)

    @pl.when(cid == 1)
    def _():
        run(gr_hbm, outr_hbm)


_seg_call = pl.kernel(
    _seg_body,
    out_type=[jax.ShapeDtypeStruct((NP, DH), jnp.float32),
              jax.ShapeDtypeStruct((NP, DH), jnp.float32)],
    mesh=_mesh,
    scratch_types=[
        pltpu.VMEM((K,), jnp.int32),
        pltpu.VMEM((K,), jnp.int32),
        pltpu.VMEM((KR,), jnp.int32),
        pltpu.VMEM((K, DH), jnp.float32),
        pltpu.VMEM((KR, DH), jnp.float32),
        pltpu.VMEM((KR, DH), jnp.float32),
        pltpu.VMEM_SHARED((NA, DH), jnp.float32),
        pltpu.SemaphoreType.DMA,
    ],
)


BN = 1000


def _mm_body(x_ref, w_ref, h_ref):
    h_ref[...] = jnp.dot(x_ref[...], w_ref[...],
                         preferred_element_type=jnp.float32,
                         precision=lax.Precision.HIGHEST)


def _mm_call(x, W):
    return pl.pallas_call(
        _mm_body,
        grid=(N // BN,),
        in_specs=[pl.BlockSpec((BN, D), lambda i: (i, 0)),
                  pl.BlockSpec((D, D), lambda i: (0, 0))],
        out_specs=pl.BlockSpec((BN, D), lambda i: (i, 0)),
        out_shape=jax.ShapeDtypeStruct((N, D), jnp.float32),
    )(x, W)


def _scale_body(h_ref, deg_ref, gl_ref, gr_ref, dinv_ref):
    dinv = lax.rsqrt(deg_ref[...] + 1.0)
    g = h_ref[...] * dinv
    gl_ref[...] = g[:, :DH]
    gr_ref[...] = g[:, DH:]
    dinv_ref[...] = dinv


def _scale_call(h, deg):
    return pl.pallas_call(
        _scale_body,
        grid=(N // BN,),
        in_specs=[pl.BlockSpec((BN, D), lambda i: (i, 0)),
                  pl.BlockSpec((BN, 1), lambda i: (i, 0))],
        out_specs=[pl.BlockSpec((BN, DH), lambda i: (i, 0)),
                   pl.BlockSpec((BN, DH), lambda i: (i, 0)),
                   pl.BlockSpec((BN, 1), lambda i: (i, 0))],
        out_shape=[jax.ShapeDtypeStruct((N, DH), jnp.float32),
                   jax.ShapeDtypeStruct((N, DH), jnp.float32),
                   jax.ShapeDtypeStruct((N, 1), jnp.float32)],
    )(h, deg)


def _ep_body(sl_ref, sr_ref, gl_ref, gr_ref, dinv_ref, x_ref,
             b_ref, gam_ref, bet_ref, o_ref):
    s = jnp.concatenate([sl_ref[...] + gl_ref[...],
                         sr_ref[...] + gr_ref[...]], axis=1)
    agg = s * dinv_ref[...] + b_ref[...]
    mu = jnp.mean(agg, axis=1, keepdims=True)
    dlt = agg - mu
    var = jnp.mean(dlt * dlt, axis=1, keepdims=True)
    hn = dlt * lax.rsqrt(var + EPS) * gam_ref[...] + bet_ref[...]
    o_ref[...] = jnp.maximum(hn, 0.0) + x_ref[...]


def _ep_call(sl, sr, gl, gr, dinv, x, b, gam, bet):
    half = pl.BlockSpec((BN, DH), lambda i: (i, 0))
    vec = pl.BlockSpec((1, D), lambda i: (0, 0))
    return pl.pallas_call(
        _ep_body,
        grid=(N // BN,),
        in_specs=[half, half, half, half,
                  pl.BlockSpec((BN, 1), lambda i: (i, 0)),
                  pl.BlockSpec((BN, D), lambda i: (i, 0)),
                  vec, vec, vec],
        out_specs=pl.BlockSpec((BN, D), lambda i: (i, 0)),
        out_shape=jax.ShapeDtypeStruct((N, D), jnp.float32),
    )(sl, sr, gl, gr, dinv, x, b, gam, bet)


def kernel(x, edge_index, W, b, ln_gamma, ln_beta):
    src = edge_index[0]
    dst = edge_index[1]
    rid = jnp.concatenate([jnp.arange(N, dtype=jnp.int32),
                           jnp.full((NP - N,), N, dtype=jnp.int32)])
    ones16 = jnp.ones((K, L), jnp.float32)
    zer16 = jnp.zeros((KR, L), jnp.float32)
    zer128 = jnp.zeros((KR, DH), jnp.float32)
    hist = _deg_call(dst, rid, ones16, zer16)
    deg = hist[:N, 0:1]
    h = _mm_call(x, W)
    gl, gr, dinv = _scale_call(h, deg)
    slp, srp = _seg_call(gl, gr, src, dst, rid, zer128)
    return _ep_call(slp[:N], srp[:N], gl, gr, dinv, x,
                    b.reshape(1, D), ln_gamma.reshape(1, D),
                    ln_beta.reshape(1, D))


# speedup vs baseline: 14.9079x; 1.6589x over previous
import jax
import jax.numpy as jnp
from jax import lax
from jax.experimental import pallas as pl
from jax.experimental.pallas import tpu as pltpu
from jax.experimental.pallas import tpu_sc as plsc

N = 10000
E = 160000
D = 256
DH = D // 2
EPS = 1e-5

NS = 16
L = 16
KC = 64
EC = 163840
EROWS = EC // KC
CH2 = EC // NS // KC
CHA = EC // 2 // NS // KC
NP = 10240
RPT = NP // NS
KR = 64
RCH = RPT // KR
NA = N + L

_mesh = plsc.VectorSubcoreMesh(core_axis_name="c", subcore_axis_name="s")


def _deg_body(dst2_hbm, rid_hbm, ones_hbm, zer_hbm, out0_hbm, out1_hbm,
              idxb_v, ridx_v, ones_v, zbuf_v, gbuf_v, hist_sh, sem):
    cid = lax.axis_index("c")
    sid = lax.axis_index("s")

    def run(out_hbm):
        pltpu.sync_copy(ones_hbm, ones_v)
        pltpu.sync_copy(zer_hbm, zbuf_v)
        pltpu.sync_copy(
            dst2_hbm.at[pl.ds(cid * (NS * CHA) + sid * CHA, CHA)], idxb_v)

        @pl.loop(0, RCH)
        def _(j):
            off = pl.multiple_of(sid * RPT + j * KR, 8)
            pltpu.sync_copy(rid_hbm.at[pl.ds(off, KR)], ridx_v)
            pltpu.sync_copy(zbuf_v, hist_sh.at[ridx_v])

        plsc.subcore_barrier()

        @pl.loop(0, CHA)
        def _(s):
            pltpu.sync_copy(ones_v, hist_sh.at[idxb_v.at[s]], add=True)

        plsc.subcore_barrier()

        @pl.loop(0, RCH)
        def _(j):
            off = pl.multiple_of(sid * RPT + j * KR, 8)
            pltpu.sync_copy(rid_hbm.at[pl.ds(off, KR)], ridx_v)
            pltpu.sync_copy(hist_sh.at[ridx_v], gbuf_v)
            pltpu.sync_copy(gbuf_v, out_hbm.at[pl.ds(off, KR)])

    @pl.when(cid == 0)
    def _():
        run(out0_hbm)

    @pl.when(cid == 1)
    def _():
        run(out1_hbm)


_deg_call = pl.kernel(
    _deg_body,
    out_type=[jax.ShapeDtypeStruct((NP, L), jnp.float32),
              jax.ShapeDtypeStruct((NP, L), jnp.float32)],
    mesh=_mesh,
    scratch_types=[
        pltpu.VMEM((CHA, KC), jnp.int32),
        pltpu.VMEM((KR,), jnp.int32),
        pltpu.VMEM((KC, L), jnp.float32),
        pltpu.VMEM((KR, L), jnp.float32),
        pltpu.VMEM((KR, L), jnp.float32),
        pltpu.VMEM_SHARED((NA, L), jnp.float32),
        pltpu.SemaphoreType.DMA,
    ],
)


def _seg_body(gl_hbm, gr_hbm, src2_hbm, dst2_hbm, rid_hbm, zer_hbm,
              outl_hbm, outr_hbm,
              dstb_v, sidx_a, sidx_b, ridx_v, rows_a, rows_b,
              acc_sh, semg_a, semg_b, semi_a, semi_b):
    cid = lax.axis_index("c")
    sid = lax.axis_index("s")
    tbase = sid * CH2

    def run(g_hbm, out_hbm):
        pltpu.sync_copy(zer_hbm, rows_a)
        pltpu.sync_copy(dst2_hbm.at[pl.ds(tbase, CH2)], dstb_v)

        @pl.loop(0, RCH)
        def _(j):
            off = pl.multiple_of(sid * RPT + j * KR, 8)
            pltpu.sync_copy(rid_hbm.at[pl.ds(off, KR)], ridx_v)
            pltpu.sync_copy(rows_a, acc_sh.at[ridx_v])

        plsc.subcore_barrier()

        pltpu.sync_copy(src2_hbm.at[tbase], sidx_a)
        pltpu.async_copy(g_hbm.at[sidx_a], rows_a, semg_a)
        pltpu.async_copy(src2_hbm.at[tbase + 1], sidx_b, semi_b)

        def _wait_g(rows, sem):
            pltpu.make_async_copy(g_hbm.at[pl.ds(0, KC)], rows, sem).wait()

        def _wait_i(sidx, sem):
            pltpu.make_async_copy(src2_hbm.at[0], sidx, sem).wait()

        @pl.loop(0, CH2 // 2)
        def _(s):
            a = s * 2
            _wait_g(rows_a, semg_a)
            nxt_a = jnp.where(a + 2 < CH2, tbase + a + 2, tbase)
            pltpu.async_copy(src2_hbm.at[nxt_a], sidx_a, semi_a)
            _wait_i(sidx_b, semi_b)
            pltpu.async_copy(g_hbm.at[sidx_b], rows_b, semg_b)
            pltpu.sync_copy(rows_a, acc_sh.at[dstb_v.at[a]], add=True)
            _wait_i(sidx_a, semi_a)
            pltpu.async_copy(g_hbm.at[sidx_a], rows_a, semg_a)
            _wait_g(rows_b, semg_b)
            pltpu.sync_copy(rows_b, acc_sh.at[dstb_v.at[a + 1]], add=True)
            nxt_b = jnp.where(a + 3 < CH2, tbase + a + 3, tbase)
            pltpu.async_copy(src2_hbm.at[nxt_b], sidx_b, semi_b)

        _wait_g(rows_a, semg_a)
        _wait_i(sidx_b, semi_b)

        plsc.subcore_barrier()

        @pl.loop(0, RCH)
        def _(j):
            off = pl.multiple_of(sid * RPT + j * KR, 8)
            pltpu.sync_copy(rid_hbm.at[pl.ds(off, KR)], ridx_v)
            pltpu.sync_copy(acc_sh.at[ridx_v], rows_b)
            pltpu.sync_copy(rows_b, out_hbm.at[pl.ds(off, KR)])

    @pl.when(cid == 0)
    def _():
        run(gl_hbm, outl_hbm)

    @pl.when(cid == 1)
    def _():
        run(gr_hbm, outr_hbm)


_seg_call = pl.kernel(
    _seg_body,
    out_type=[jax.ShapeDtypeStruct((NP, DH), jnp.float32),
              jax.ShapeDtypeStruct((NP, DH), jnp.float32)],
    mesh=_mesh,
    scratch_types=[
        pltpu.VMEM((CH2, KC), jnp.int32),
        pltpu.VMEM((KC,), jnp.int32),
        pltpu.VMEM((KC,), jnp.int32),
        pltpu.VMEM((KR,), jnp.int32),
        pltpu.VMEM((KC, DH), jnp.float32),
        pltpu.VMEM((KC, DH), jnp.float32),
        pltpu.VMEM_SHARED((NA, DH), jnp.float32),
        pltpu.SemaphoreType.DMA,
        pltpu.SemaphoreType.DMA,
        pltpu.SemaphoreType.DMA,
        pltpu.SemaphoreType.DMA,
    ],
)


BN = 1000


def _mm_body(x_ref, w_ref, h_ref):
    h_ref[...] = jnp.dot(x_ref[...], w_ref[...],
                         preferred_element_type=jnp.float32,
                         precision=lax.Precision.HIGHEST)


def _mm_call(x, W):
    return pl.pallas_call(
        _mm_body,
        grid=(N // BN,),
        in_specs=[pl.BlockSpec((BN, D), lambda i: (i, 0)),
                  pl.BlockSpec((D, D), lambda i: (0, 0))],
        out_specs=pl.BlockSpec((BN, D), lambda i: (i, 0)),
        out_shape=jax.ShapeDtypeStruct((N, D), jnp.float32),
    )(x, W)


def _scale_body(h_ref, d0_ref, d1_ref, gl_ref, gr_ref, dinv_ref):
    dinv = lax.rsqrt(d0_ref[...] + d1_ref[...] + 1.0)
    g = h_ref[...] * dinv
    gl_ref[...] = g[:, :DH]
    gr_ref[...] = g[:, DH:]
    dinv_ref[...] = dinv


def _scale_call(h, d0, d1):
    col = pl.BlockSpec((BN, 1), lambda i: (i, 0))
    return pl.pallas_call(
        _scale_body,
        grid=(N // BN,),
        in_specs=[pl.BlockSpec((BN, D), lambda i: (i, 0)), col, col],
        out_specs=[pl.BlockSpec((BN, DH), lambda i: (i, 0)),
                   pl.BlockSpec((BN, DH), lambda i: (i, 0)),
                   col],
        out_shape=[jax.ShapeDtypeStruct((N, DH), jnp.float32),
                   jax.ShapeDtypeStruct((N, DH), jnp.float32),
                   jax.ShapeDtypeStruct((N, 1), jnp.float32)],
    )(h, d0, d1)


def _ep_body(sl_ref, sr_ref, gl_ref, gr_ref, dinv_ref, x_ref,
             b_ref, gam_ref, bet_ref, o_ref):
    s = jnp.concatenate([sl_ref[...] + gl_ref[...],
                         sr_ref[...] + gr_ref[...]], axis=1)
    agg = s * dinv_ref[...] + b_ref[...]
    mu = jnp.mean(agg, axis=1, keepdims=True)
    dlt = agg - mu
    var = jnp.mean(dlt * dlt, axis=1, keepdims=True)
    hn = dlt * lax.rsqrt(var + EPS) * gam_ref[...] + bet_ref[...]
    o_ref[...] = jnp.maximum(hn, 0.0) + x_ref[...]


def _ep_call(sl, sr, gl, gr, dinv, x, b, gam, bet):
    half = pl.BlockSpec((BN, DH), lambda i: (i, 0))
    vec = pl.BlockSpec((1, D), lambda i: (0, 0))
    return pl.pallas_call(
        _ep_body,
        grid=(N // BN,),
        in_specs=[half, half, half, half,
                  pl.BlockSpec((BN, 1), lambda i: (i, 0)),
                  pl.BlockSpec((BN, D), lambda i: (i, 0)),
                  vec, vec, vec],
        out_specs=pl.BlockSpec((BN, D), lambda i: (i, 0)),
        out_shape=jax.ShapeDtypeStruct((N, D), jnp.float32),
    )(sl, sr, gl, gr, dinv, x, b, gam, bet)


def kernel(x, edge_index, W, b, ln_gamma, ln_beta):
    src = edge_index[0]
    dst = edge_index[1]
    npad = EC - E
    pad_i = jnp.arange(npad, dtype=jnp.int32)
    src2 = jnp.concatenate([src, (pad_i * 8) % N]).reshape(EROWS, KC)
    dst2 = jnp.concatenate([dst, N + (pad_i % L)]).reshape(EROWS, KC)
    rid = jnp.concatenate([jnp.arange(N, dtype=jnp.int32),
                           N + (jnp.arange(NP - N, dtype=jnp.int32) % L)])
    ones16 = jnp.ones((KC, L), jnp.float32)
    zer16 = jnp.zeros((KR, L), jnp.float32)
    zer128 = jnp.zeros((KR, DH), jnp.float32)
    h0, h1 = _deg_call(dst2, rid, ones16, zer16)
    d0, d1 = h0[:N, 0:1], h1[:N, 0:1]
    h = _mm_call(x, W)
    gl, gr, dinv = _scale_call(h, d0, d1)
    slp, srp = _seg_call(gl, gr, src2, dst2, rid, zer128)
    return _ep_call(slp[:N], srp[:N], gl, gr, dinv, x,
                    b.reshape(1, D), ln_gamma.reshape(1, D),
                    ln_beta.reshape(1, D))
